# baseline (speedup 1.0000x reference)
import functools

import jax
import jax.numpy as jnp
from jax import lax
from jax.experimental import pallas as pl
from jax.experimental.pallas import tpu as pltpu
from jax.experimental.pallas import tpu_sc as plsc

_B = 64
_C = 768
_HW = 1024
_R = _B * _C
_NC = 2
_NS = 16
_NW = _NC * _NS
_RPW = _R // _NW
_CH = 64
_NCH = _RPW // _CH


def _sc_gather(xf, idx3):
    mesh = plsc.VectorSubcoreMesh(core_axis_name="c", subcore_axis_name="s")

    @functools.partial(
        pl.kernel,
        mesh=mesh,
        out_type=jax.ShapeDtypeStruct((_R, _HW), jnp.float32),
        scratch_types=[
            pltpu.VMEM((_NCH, _CH), jnp.int32),
            pltpu.VMEM((_CH, _HW), jnp.float32),
            pltpu.SemaphoreType.DMA,
        ],
    )
    def k(x_hbm, idx_hbm, out_hbm, idx_v, rows_v, sem):
        wid = lax.axis_index("s") * _NC + lax.axis_index("c")
        pltpu.sync_copy(idx_hbm.at[wid], idx_v)
        base = wid * _RPW

        def body(j, carry):
            pltpu.async_copy(x_hbm.at[idx_v.at[j]], rows_v, sem).wait()
            pltpu.sync_copy(rows_v, out_hbm.at[pl.ds(base + j * _CH, _CH)])
            return carry

        lax.fori_loop(0, _NCH, body, 0)

    return k(xf, idx3)


def kernel(x, objective, z_list, indices):
    xf = x.reshape(_R, _HW)
    idx3 = (
        jnp.arange(_B, dtype=jnp.int32)[:, None] * _C + indices[None, :]
    ).reshape(_NW, _NCH, _CH)
    yf = _sc_gather(xf, idx3)
    return (yf.reshape(_B, _C, 32, 32), objective, z_list)


# speedup vs baseline: 1.0111x; 1.0111x over previous
import functools

import jax
import jax.numpy as jnp
from jax import lax
from jax.experimental import pallas as pl
from jax.experimental.pallas import tpu as pltpu
from jax.experimental.pallas import tpu_sc as plsc

_B = 64
_C = 768
_HW = 1024
_R = _B * _C
_NC = 2
_NS = 16
_NW = _NC * _NS
_RPW = _R // _NW
_CH = 48
_NCH = _RPW // _CH
_NBUF = 2


def _sc_gather(xf, idx3):
    mesh = plsc.VectorSubcoreMesh(core_axis_name="c", subcore_axis_name="s")

    @functools.partial(
        pl.kernel,
        mesh=mesh,
        out_type=jax.ShapeDtypeStruct((_R, _HW), jnp.float32),
        scratch_types=[
            pltpu.VMEM((_NCH, _CH), jnp.int32),
            pltpu.VMEM((_CH, _HW), jnp.float32),
            pltpu.VMEM((_CH, _HW), jnp.float32),
            pltpu.SemaphoreType.DMA,
            pltpu.SemaphoreType.DMA,
            pltpu.SemaphoreType.DMA,
            pltpu.SemaphoreType.DMA,
        ],
    )
    def k(x_hbm, idx_hbm, out_hbm, idx_v, rows0, rows1, g0, g1, s0, s1):
        wid = lax.axis_index("s") * _NC + lax.axis_index("c")
        pltpu.sync_copy(idx_hbm.at[wid], idx_v)
        base = wid * _RPW
        rows = (rows0, rows1)
        gsem = (g0, g1)
        ssem = (s0, s1)

        def start_gather(j):
            p = j % _NBUF
            return pltpu.async_copy(x_hbm.at[idx_v.at[j]], rows[p], gsem[p])

        def start_scatter(j):
            p = j % _NBUF
            return pltpu.async_copy(
                rows[p], out_hbm.at[pl.ds(base + j * _CH, _CH)], ssem[p]
            )

        g = {0: start_gather(0), 1: start_gather(1)}
        s = {}
        for j in range(_NCH):
            g[j].wait()
            s[j] = start_scatter(j)
            if j + _NBUF < _NCH:
                s[j].wait()
                g[j + _NBUF] = start_gather(j + _NBUF)
        for j in range(_NCH - _NBUF, _NCH):
            s[j].wait()

    return k(xf, idx3)


def kernel(x, objective, z_list, indices):
    xf = x.reshape(_R, _HW)
    idx3 = (
        jnp.arange(_B, dtype=jnp.int32)[:, None] * _C + indices[None, :]
    ).reshape(_NW, _NCH, _CH)
    yf = _sc_gather(xf, idx3)
    return (yf.reshape(_B, _C, 32, 32), objective, z_list)
